# Initial kernel scaffold; baseline (speedup 1.0000x reference)
import jax
import jax.numpy as jnp
from jax.experimental import pallas as pl


def kernel(x, edge_index, W_l1, b_l1, W_r1, W_l2, b_l2, W_r2, W_l3, b_l3, W_r3, T1a_W, T1a_b, T2a_W, T2a_b, T1b_W, T1b_b, T2b_W, T2b_b, lin2_W, lin2_b, lin1_W, lin1_b):
    raise NotImplementedError("write your pallas kernel here")


# speedup vs baseline: 16.8476x; 16.8476x over previous
import functools

import jax
import jax.numpy as jnp
from jax import lax
from jax.experimental import pallas as pl
from jax.experimental.pallas import tpu as pltpu
from jax.experimental.pallas import tpu_sc as plsc

N = 50000
E = 800000
NC = 2
NS = 16
NW = NC * NS
VP = 50048
EW = E // NW
STR = VP // NS

F32 = jnp.float32


def _mesh():
  return plsc.VectorSubcoreMesh(core_axis_name="c", subcore_axis_name="s")


@functools.lru_cache(maxsize=None)
def _make_hop(D, chunk):
  iters = EW // chunk
  vec = D == 0
  tab_s = (VP,) if vec else (VP, D)
  row_s = (chunk,) if vec else (chunk, D)
  if vec:
    out_t = jax.ShapeDtypeStruct((NC * VP,), F32)
  else:
    out_t = (jax.ShapeDtypeStruct((VP, D), F32),
             jax.ShapeDtypeStruct((VP, D), F32))

  @functools.partial(
      pl.kernel,
      mesh=_mesh(),
      compiler_params=pltpu.CompilerParams(use_tc_tiling_on_sc=False),
      out_type=out_t,
      scratch_types=[
          pltpu.VMEM((chunk,), jnp.int32),
          pltpu.VMEM((chunk,), jnp.int32),
          pltpu.VMEM(row_s, F32),
          pltpu.VMEM_SHARED(tab_s, F32),
          pltpu.SemaphoreType.DMA,
      ],
  )
  def hop(table, src, dst, zeros, *rest):
    if vec:
      out, src_v, dst_v, rows_v, acc, sem = rest
    else:
      out0, out1, src_v, dst_v, rows_v, acc, sem = rest
    c = lax.axis_index("c")
    s = lax.axis_index("s")
    r0 = s * STR
    sub = STR if chunk >= STR else 184
    for j in range(STR // sub):
      o = r0 + j * sub
      pltpu.sync_copy(zeros.at[pl.ds(o, sub)], rows_v.at[pl.ds(0, sub)])
      pltpu.sync_copy(rows_v.at[pl.ds(0, sub)], acc.at[pl.ds(o, sub)])
    plsc.subcore_barrier()
    base = (s * NC + c) * EW
    for t in range(iters):
      off = base + t * chunk
      pltpu.sync_copy(src.at[pl.ds(off, chunk)], src_v)
      pltpu.sync_copy(dst.at[pl.ds(off, chunk)], dst_v)
      pltpu.async_copy(table.at[src_v], rows_v, sem).wait()
      pltpu.sync_copy(rows_v, acc.at[dst_v], add=True)
    plsc.subcore_barrier()
    for j in range(STR // sub):
      o = r0 + j * sub
      pltpu.sync_copy(acc.at[pl.ds(o, sub)], rows_v.at[pl.ds(0, sub)])
      if vec:
        pltpu.sync_copy(rows_v.at[pl.ds(0, sub)],
                        out.at[pl.ds(c * VP + o, sub)])
      else:
        @pl.when(c == 0)
        def _():
          pltpu.sync_copy(rows_v.at[pl.ds(0, sub)], out0.at[pl.ds(o, sub)])

        @pl.when(c == 1)
        def _():
          pltpu.sync_copy(rows_v.at[pl.ds(0, sub)], out1.at[pl.ds(o, sub)])

  return hop


B = 544
G = VP // B
FR = VP // 128


def _rows(k):
  return pl.BlockSpec((B, k), lambda i: (i, 0))


def _full(shape):
  return pl.BlockSpec(shape, lambda i: (0,) * len(shape))


def _tc(body, ins, in_specs, out_shapes, out_specs, grid=None):
  return pl.pallas_call(
      body,
      grid=(G,) if grid is None else grid,
      in_specs=in_specs,
      out_specs=out_specs,
      out_shape=out_shapes,
  )(*ins)


def _prep_body(d0, d1, x, dis_o, dis2_o, invmd_o, sx_o):
  deg = d0[...] + d1[...]
  md = jnp.maximum(deg, 1.0)
  dis = jnp.where(deg > 0, lax.rsqrt(md), 0.0)
  dis_o[...] = dis
  dis2_o[...] = dis * dis
  invmd_o[...] = 1.0 / md
  sx_o[...] = x[...] * dis


def _sage1_body(q0, q1, x, invmd, wl, wr, b, x1_o):
  mean = (q0[...] + q1[...]) * invmd[...]
  x1_o[...] = jnp.maximum(
      jnp.dot(mean, wl[...], preferred_element_type=F32)
      + jnp.dot(x[...], wr[...], preferred_element_type=F32) + b[...], 0.0)


def _sage2_body(*refs):
  qs = refs[:32]
  x1, invmd, wl, wr, b, wl3, x12_o, y3_o = refs[32:]
  w = wl[...]
  im = invmd[...]
  acc = jnp.dot(x1[...], wr[...], preferred_element_type=F32) + b[...]
  for p in range(16):
    u = (qs[2 * p][...] + qs[2 * p + 1][...]) * im
    acc = acc + jnp.dot(u, w[8 * p:8 * p + 8, :], preferred_element_type=F32)
  x12 = jnp.maximum(acc, 0.0)
  x12_o[...] = x12
  y3_o[...] = jnp.dot(x12, wl3[...], preferred_element_type=F32)


def _tagscale_body(q0, q1, dis2, tbl_o):
  tbl_o[...] = dis2[...] * (q0[...] + q1[...])


def _tagdense_body(*refs):
  (qt, x, dis, wa, ba, wb, bb, w2a, w2b,
   za_o, sa_o, zb_o, sb_o) = (refs[:18], *refs[18:])
  ua = jnp.zeros((B, 8), F32)
  ub = jnp.zeros((B, 8), F32)
  for k in range(9):
    u = qt[2 * k][...] + qt[2 * k + 1][...]
    if k < 3:
      ua = ua + jnp.dot(u, wa[k + 1], preferred_element_type=F32)
    ub = ub + jnp.dot(u, wb[k + 1], preferred_element_type=F32)
  d = dis[...]
  x2 = jnp.maximum(
      jnp.dot(x[...], wa[0], preferred_element_type=F32) + d * ua + ba[...],
      0.0)
  x3 = jnp.maximum(
      jnp.dot(x[...], wb[0], preferred_element_type=F32) + d * ub + bb[...],
      0.0)
  za_o[...] = jnp.dot(x2, w2a[...][:, :1], preferred_element_type=F32)
  sa_o[...] = jnp.dot(x2, w2a[...][:, 1:], preferred_element_type=F32) * d
  zb_o[...] = jnp.dot(x3, w2b[...][:, :1], preferred_element_type=F32)
  sb_o[...] = jnp.dot(x3, w2b[...][:, 1:], preferred_element_type=F32) * d


def _hglue_body(sa, q0, q1, dis2, tbl_o):
  tbl_o[...] = sa[...] + dis2[...] * (q0[...] + q1[...])


def _flat(a):
  return a.reshape(FR, 128)


def _final_body(x12, q30, q31, qa0, qa1, qb0, qb1, za, zb, dis, invmd,
                wr3, bl3, b2a, b2b, l2w, l2b, l1w, l1b, out_o):
  d = dis[...]
  x13 = jnp.maximum(
      (q30[...] + q31[...]) * invmd[...] + bl3[...]
      + jnp.dot(x12[...], wr3[...], preferred_element_type=F32), 0.0)
  x2f = jnp.maximum(za[...] + d * (qa0[...] + qa1[...]) + b2a[...], 0.0)
  x3f = jnp.maximum(zb[...] + d * (qb0[...] + qb1[...]) + b2b[...], 0.0)
  w2 = l2w[...]
  x23 = jnp.maximum(x2f * w2[0:1, :] + x3f * w2[1:2, :] + l2b[...], 0.0)
  w1 = l1w[...]
  out_o[...] = jnp.maximum(x13 * w1[0:1, :] + x23 * w1[1:2, :] + l1b[...],
                           0.0)


def kernel(x, edge_index, W_l1, b_l1, W_r1, W_l2, b_l2, W_r2, W_l3, b_l3,
           W_r3, T1a_W, T1a_b, T2a_W, T2a_b, T1b_W, T1b_b, T2b_W, T2b_b,
           lin2_W, lin2_b, lin1_W, lin1_b):
  xp = jnp.pad(x, ((0, VP - N), (0, 0)))
  src = edge_index[0]
  dst = edge_index[1]
  z1 = jnp.zeros((VP,), F32)
  z8 = jnp.zeros((VP, 8), F32)

  ones_vp = jnp.ones((VP,), F32)

  hop1 = _make_hop(0, 25000)
  hop8 = _make_hop(8, 5000)

  degp = hop1(ones_vp, src, dst, z1).reshape(NC, VP)
  d0 = degp[0].reshape(VP, 1)
  d1 = degp[1].reshape(VP, 1)

  dis, dis2, invmd, sx = _tc(
      _prep_body,
      [d0, d1, xp],
      [_rows(1), _rows(1), _rows(8)],
      [jax.ShapeDtypeStruct((VP, 1), F32)] * 3
      + [jax.ShapeDtypeStruct((VP, 8), F32)],
      [_rows(1), _rows(1), _rows(1), _rows(8)],
  )

  q8 = hop8(xp, src, dst, z8)
  x1 = _tc(
      _sage1_body,
      [q8[0], q8[1], xp, invmd, W_l1, W_r1, b_l1.reshape(1, 128)],
      [_rows(8), _rows(8), _rows(8), _rows(1), _full((8, 128)),
       _full((8, 128)), _full((1, 128))],
      jax.ShapeDtypeStruct((VP, 128), F32),
      _rows(128),
  )

  x1t = x1.reshape(VP, 16, 8).transpose(1, 0, 2)
  qs = [hop8(x1t[p], src, dst, z8) for p in range(16)]
  qflat = [q for pair in qs for q in pair]
  x12, y3 = _tc(
      _sage2_body,
      qflat + [x1, invmd, W_l2, W_r2, b_l2.reshape(1, 128), W_l3],
      [_rows(8)] * 32 + [_rows(128), _rows(1), _full((128, 128)),
                         _full((128, 128)), _full((1, 128)),
                         _full((128, 1))],
      [jax.ShapeDtypeStruct((VP, 128), F32),
       jax.ShapeDtypeStruct((VP, 1), F32)],
      [_rows(128), _rows(1)],
  )

  q3 = hop1(y3.reshape(VP), src, dst, z1).reshape(NC, VP)

  qt = []
  tbl = sx
  for k in range(9):
    qk = hop8(tbl, src, dst, z8)
    qt.append(qk)
    if k < 8:
      tbl = _tc(
          _tagscale_body,
          [qk[0], qk[1], dis2],
          [_rows(8), _rows(8), _rows(1)],
          jax.ShapeDtypeStruct((VP, 8), F32),
          _rows(8),
      )

  w2a = T2a_W[..., 0].T
  w2b = T2b_W[..., 0].T
  qt_flat = []
  for qk in qt:
    qt_flat += [qk[0], qk[1]]
  za, sa, zb, sb = _tc(
      _tagdense_body,
      qt_flat + [xp, dis, T1a_W, T1a_b.reshape(1, 8), T1b_W,
                 T1b_b.reshape(1, 8), w2a, w2b],
      [_rows(8)] * 18 + [_rows(8), _rows(1), _full((4, 8, 8)),
                         _full((1, 8)), _full((10, 8, 8)), _full((1, 8)),
                         _full((8, 4)), _full((8, 10))],
      [jax.ShapeDtypeStruct((VP, 1), F32),
       jax.ShapeDtypeStruct((VP, 3), F32),
       jax.ShapeDtypeStruct((VP, 1), F32),
       jax.ShapeDtypeStruct((VP, 9), F32)],
      [_rows(1), _rows(3), _rows(1), _rows(9)],
  )

  def horner(scols, nsteps):
    tblh = scols[:, nsteps - 1:nsteps]
    q = None
    for j in range(nsteps - 1, -1, -1):
      q = hop1(tblh.reshape(VP), src, dst, z1).reshape(NC, VP)
      if j > 0:
        tblh = _tc(
            _hglue_body,
            [_flat(scols[:, j - 1:j]), _flat(q[0]), _flat(q[1]),
             _flat(dis2)],
            [_full((FR, 128))] * 4,
            jax.ShapeDtypeStruct((FR, 128), F32),
            _full((FR, 128)),
            grid=(1,),
        )
    return q

  qa = horner(sa, 3)
  qb = horner(sb, 9)

  out = _tc(
      _final_body,
      [x12, q3[0].reshape(VP, 1), q3[1].reshape(VP, 1),
       qa[0].reshape(VP, 1), qa[1].reshape(VP, 1),
       qb[0].reshape(VP, 1), qb[1].reshape(VP, 1),
       za, zb, dis, invmd, W_r3, b_l3.reshape(1, 1),
       T2a_b.reshape(1, 1), T2b_b.reshape(1, 1), lin2_W,
       lin2_b.reshape(1, 1), lin1_W, lin1_b.reshape(1, 1)],
      [_rows(128)] + [_rows(1)] * 10
      + [_full((128, 1)), _full((1, 1)), _full((1, 1)), _full((1, 1)),
         _full((2, 1)), _full((1, 1)), _full((2, 1)), _full((1, 1))],
      jax.ShapeDtypeStruct((VP, 1), F32),
      _rows(1),
  )
  return out[:N]


# speedup vs baseline: 19.0809x; 1.1326x over previous
import functools

import jax
import jax.numpy as jnp
from jax import lax
from jax.experimental import pallas as pl
from jax.experimental.pallas import tpu as pltpu
from jax.experimental.pallas import tpu_sc as plsc

N = 50000
E = 800000
NC = 2
NS = 16
NW = NC * NS
VP = 50048
EW = E // NW
STR = VP // NS

F32 = jnp.float32


def _mesh():
  return plsc.VectorSubcoreMesh(core_axis_name="c", subcore_axis_name="s")


@functools.lru_cache(maxsize=None)
def _make_hop(D, chunk):
  iters = EW // chunk
  vec = D == 0
  tab_s = (VP,) if vec else (VP, D)
  row_s = (chunk,) if vec else (chunk, D)
  if vec:
    out_t = jax.ShapeDtypeStruct((NC * VP,), F32)
  else:
    out_t = (jax.ShapeDtypeStruct((VP, D), F32),
             jax.ShapeDtypeStruct((VP, D), F32))

  @functools.partial(
      pl.kernel,
      mesh=_mesh(),
      compiler_params=pltpu.CompilerParams(use_tc_tiling_on_sc=False),
      out_type=out_t,
      scratch_types=[
          pltpu.VMEM((chunk,), jnp.int32),
          pltpu.VMEM((chunk,), jnp.int32),
          pltpu.VMEM((chunk,), jnp.int32),
          pltpu.VMEM((chunk,), jnp.int32),
          pltpu.VMEM(row_s, F32),
          pltpu.VMEM(row_s, F32),
          pltpu.VMEM_SHARED(tab_s, F32),
          pltpu.SemaphoreType.DMA,
          pltpu.SemaphoreType.DMA,
          pltpu.SemaphoreType.DMA,
      ],
  )
  def hop(table, src, dst, zeros, *rest):
    if vec:
      (out, src_v0, src_v1, dst_v0, dst_v1, rows_v0, rows_v1, acc,
       sem_i0, sem_i1, sem_g) = rest
    else:
      (out0, out1, src_v0, src_v1, dst_v0, dst_v1, rows_v0, rows_v1, acc,
       sem_i0, sem_i1, sem_g) = rest
    srcb = (src_v0, src_v1)
    dstb = (dst_v0, dst_v1)
    rowb = (rows_v0, rows_v1)
    semi = (sem_i0, sem_i1)
    c = lax.axis_index("c")
    s = lax.axis_index("s")
    r0 = s * STR
    sub = STR if chunk >= STR else 184
    for j in range(STR // sub):
      o = r0 + j * sub
      pltpu.sync_copy(zeros.at[pl.ds(o, sub)], rows_v0.at[pl.ds(0, sub)])
      pltpu.sync_copy(rows_v0.at[pl.ds(0, sub)], acc.at[pl.ds(o, sub)])
    plsc.subcore_barrier()
    base = (s * NC + c) * EW

    def start_idx(t):
      b = t % 2
      off = base + t * chunk
      d1 = pltpu.async_copy(src.at[pl.ds(off, chunk)], srcb[b], semi[b])
      d2 = pltpu.async_copy(dst.at[pl.ds(off, chunk)], dstb[b], semi[b])
      return (d1, d2)

    idx_d = [None] * iters
    gat_d = [None] * iters
    idx_d[0] = start_idx(0)
    idx_d[0][0].wait()
    idx_d[0][1].wait()
    gat_d[0] = pltpu.async_copy(table.at[srcb[0]], rowb[0], sem_g)
    for t in range(iters):
      cur = t % 2
      nxt = 1 - cur
      if t + 1 < iters:
        idx_d[t + 1] = start_idx(t + 1)
      gat_d[t].wait()
      if t + 1 < iters:
        idx_d[t + 1][0].wait()
        idx_d[t + 1][1].wait()
        gat_d[t + 1] = pltpu.async_copy(table.at[srcb[nxt]], rowb[nxt],
                                        sem_g)
      pltpu.sync_copy(rowb[cur], acc.at[dstb[cur]], add=True)
    plsc.subcore_barrier()
    for j in range(STR // sub):
      o = r0 + j * sub
      pltpu.sync_copy(acc.at[pl.ds(o, sub)], rows_v0.at[pl.ds(0, sub)])
      if vec:
        pltpu.sync_copy(rows_v0.at[pl.ds(0, sub)],
                        out.at[pl.ds(c * VP + o, sub)])
      else:
        @pl.when(c == 0)
        def _():
          pltpu.sync_copy(rows_v0.at[pl.ds(0, sub)], out0.at[pl.ds(o, sub)])

        @pl.when(c == 1)
        def _():
          pltpu.sync_copy(rows_v0.at[pl.ds(0, sub)], out1.at[pl.ds(o, sub)])

  return hop


B = 544
G = VP // B
FR = VP // 128


def _rows(k):
  return pl.BlockSpec((B, k), lambda i: (i, 0))


def _full(shape):
  return pl.BlockSpec(shape, lambda i: (0,) * len(shape))


def _tc(body, ins, in_specs, out_shapes, out_specs, grid=None):
  return pl.pallas_call(
      body,
      grid=(G,) if grid is None else grid,
      in_specs=in_specs,
      out_specs=out_specs,
      out_shape=out_shapes,
  )(*ins)


def _prep_body(d0, d1, x, dis_o, dis2_o, invmd_o, sx_o):
  deg = d0[...] + d1[...]
  md = jnp.maximum(deg, 1.0)
  dis = jnp.where(deg > 0, lax.rsqrt(md), 0.0)
  dis_o[...] = dis
  dis2_o[...] = dis * dis
  invmd_o[...] = 1.0 / md
  sx_o[...] = x[...] * dis


def _sage1_body(q0, q1, x, invmd, wl, wr, b, x1_o):
  mean = (q0[...] + q1[...]) * invmd[...]
  x1_o[...] = jnp.maximum(
      jnp.dot(mean, wl[...], preferred_element_type=F32)
      + jnp.dot(x[...], wr[...], preferred_element_type=F32) + b[...], 0.0)


def _sage2_body(*refs):
  qs = refs[:16]
  x1, invmd, wl, wr, b, wl3, x12_o, y3_o = refs[16:]
  w = wl[...]
  im = invmd[...]
  acc = jnp.dot(x1[...], wr[...], preferred_element_type=F32) + b[...]
  for p in range(8):
    u = (qs[2 * p][...] + qs[2 * p + 1][...]) * im
    acc = acc + jnp.dot(u, w[16 * p:16 * p + 16, :],
                        preferred_element_type=F32)
  x12 = jnp.maximum(acc, 0.0)
  x12_o[...] = x12
  y3_o[...] = jnp.dot(x12, wl3[...], preferred_element_type=F32)


def _tagscale_body(q0, q1, dis2, tbl_o):
  tbl_o[...] = dis2[...] * (q0[...] + q1[...])


def _tagdense_body(*refs):
  (qt, x, dis, wa, ba, wb, bb, w2a, w2b,
   za_o, sa_o, zb_o, sb_o) = (refs[:18], *refs[18:])
  ua = jnp.zeros((B, 8), F32)
  ub = jnp.zeros((B, 8), F32)
  for k in range(9):
    u = qt[2 * k][...] + qt[2 * k + 1][...]
    if k < 3:
      ua = ua + jnp.dot(u, wa[k + 1], preferred_element_type=F32)
    ub = ub + jnp.dot(u, wb[k + 1], preferred_element_type=F32)
  d = dis[...]
  x2 = jnp.maximum(
      jnp.dot(x[...], wa[0], preferred_element_type=F32) + d * ua + ba[...],
      0.0)
  x3 = jnp.maximum(
      jnp.dot(x[...], wb[0], preferred_element_type=F32) + d * ub + bb[...],
      0.0)
  za_o[...] = jnp.dot(x2, w2a[...][:, :1], preferred_element_type=F32)
  sa_o[...] = jnp.dot(x2, w2a[...][:, 1:], preferred_element_type=F32) * d
  zb_o[...] = jnp.dot(x3, w2b[...][:, :1], preferred_element_type=F32)
  sb_o[...] = jnp.dot(x3, w2b[...][:, 1:], preferred_element_type=F32) * d


def _hglue_body(sa, q0, q1, dis2, tbl_o):
  tbl_o[...] = sa[...] + dis2[...] * (q0[...] + q1[...])


def _flat(a):
  return a.reshape(FR, 128)


def _final_body(x12, q30, q31, qa0, qa1, qb0, qb1, za, zb, dis, invmd,
                wr3, bl3, b2a, b2b, l2w, l2b, l1w, l1b, out_o):
  d = dis[...]
  x13 = jnp.maximum(
      (q30[...] + q31[...]) * invmd[...] + bl3[...]
      + jnp.dot(x12[...], wr3[...], preferred_element_type=F32), 0.0)
  x2f = jnp.maximum(za[...] + d * (qa0[...] + qa1[...]) + b2a[...], 0.0)
  x3f = jnp.maximum(zb[...] + d * (qb0[...] + qb1[...]) + b2b[...], 0.0)
  w2 = l2w[...]
  x23 = jnp.maximum(x2f * w2[0:1, :] + x3f * w2[1:2, :] + l2b[...], 0.0)
  w1 = l1w[...]
  out_o[...] = jnp.maximum(x13 * w1[0:1, :] + x23 * w1[1:2, :] + l1b[...],
                           0.0)


def kernel(x, edge_index, W_l1, b_l1, W_r1, W_l2, b_l2, W_r2, W_l3, b_l3,
           W_r3, T1a_W, T1a_b, T2a_W, T2a_b, T1b_W, T1b_b, T2b_W, T2b_b,
           lin2_W, lin2_b, lin1_W, lin1_b):
  xp = jnp.pad(x, ((0, VP - N), (0, 0)))
  xp16 = jnp.pad(x, ((0, VP - N), (0, 8)))
  src = edge_index[0]
  dst = edge_index[1]
  z1 = jnp.zeros((VP,), F32)
  z16 = jnp.zeros((VP, 16), F32)

  ones_vp = jnp.ones((VP,), F32)

  hop1 = _make_hop(0, 5000)
  hop16 = _make_hop(16, 1000)

  degp = hop1(ones_vp, src, dst, z1).reshape(NC, VP)
  d0 = degp[0].reshape(VP, 1)
  d1 = degp[1].reshape(VP, 1)

  dis, dis2, invmd, sx = _tc(
      _prep_body,
      [d0, d1, xp16],
      [_rows(1), _rows(1), _rows(16)],
      [jax.ShapeDtypeStruct((VP, 1), F32)] * 3
      + [jax.ShapeDtypeStruct((VP, 16), F32)],
      [_rows(1), _rows(1), _rows(1), _rows(16)],
  )

  wl1p = jnp.pad(W_l1, ((0, 8), (0, 0)))
  wr1p = jnp.pad(W_r1, ((0, 8), (0, 0)))
  q8 = hop16(xp16, src, dst, z16)
  x1 = _tc(
      _sage1_body,
      [q8[0], q8[1], xp16, invmd, wl1p, wr1p, b_l1.reshape(1, 128)],
      [_rows(16), _rows(16), _rows(16), _rows(1), _full((16, 128)),
       _full((16, 128)), _full((1, 128))],
      jax.ShapeDtypeStruct((VP, 128), F32),
      _rows(128),
  )

  x1t = x1.reshape(VP, 8, 16).transpose(1, 0, 2)
  qs = [hop16(x1t[p], src, dst, z16) for p in range(8)]
  qflat = [q for pair in qs for q in pair]
  x12, y3 = _tc(
      _sage2_body,
      qflat + [x1, invmd, W_l2, W_r2, b_l2.reshape(1, 128), W_l3],
      [_rows(16)] * 16 + [_rows(128), _rows(1), _full((128, 128)),
                          _full((128, 128)), _full((1, 128)),
                          _full((128, 1))],
      [jax.ShapeDtypeStruct((VP, 128), F32),
       jax.ShapeDtypeStruct((VP, 1), F32)],
      [_rows(128), _rows(1)],
  )

  q3 = hop1(y3.reshape(VP), src, dst, z1).reshape(NC, VP)

  qt = []
  tbl = sx
  for k in range(9):
    qk = hop16(tbl, src, dst, z16)
    qt.append(qk)
    if k < 8:
      tbl = _tc(
          _tagscale_body,
          [qk[0], qk[1], dis2],
          [_rows(16), _rows(16), _rows(1)],
          jax.ShapeDtypeStruct((VP, 16), F32),
          _rows(16),
      )

  w2a = T2a_W[..., 0].T
  w2b = T2b_W[..., 0].T
  qt_flat = []
  for qk in qt:
    qt_flat += [qk[0], qk[1]]
  za, sa, zb, sb = _tc(
      _tagdense_body,
      qt_flat + [xp16, dis, jnp.pad(T1a_W, ((0, 0), (0, 8), (0, 0))),
                 T1a_b.reshape(1, 8),
                 jnp.pad(T1b_W, ((0, 0), (0, 8), (0, 0))),
                 T1b_b.reshape(1, 8), w2a, w2b],
      [_rows(16)] * 18 + [_rows(16), _rows(1), _full((4, 16, 8)),
                         _full((1, 8)), _full((10, 16, 8)), _full((1, 8)),
                         _full((8, 4)), _full((8, 10))],
      [jax.ShapeDtypeStruct((VP, 1), F32),
       jax.ShapeDtypeStruct((VP, 3), F32),
       jax.ShapeDtypeStruct((VP, 1), F32),
       jax.ShapeDtypeStruct((VP, 9), F32)],
      [_rows(1), _rows(3), _rows(1), _rows(9)],
  )

  def horner(scols, nsteps):
    tblh = scols[:, nsteps - 1:nsteps]
    q = None
    for j in range(nsteps - 1, -1, -1):
      q = hop1(tblh.reshape(VP), src, dst, z1).reshape(NC, VP)
      if j > 0:
        tblh = _tc(
            _hglue_body,
            [_flat(scols[:, j - 1:j]), _flat(q[0]), _flat(q[1]),
             _flat(dis2)],
            [_full((FR, 128))] * 4,
            jax.ShapeDtypeStruct((FR, 128), F32),
            _full((FR, 128)),
            grid=(1,),
        )
    return q

  qa = horner(sa, 3)
  qb = horner(sb, 9)

  out = _tc(
      _final_body,
      [x12, q3[0].reshape(VP, 1), q3[1].reshape(VP, 1),
       qa[0].reshape(VP, 1), qa[1].reshape(VP, 1),
       qb[0].reshape(VP, 1), qb[1].reshape(VP, 1),
       za, zb, dis, invmd, W_r3, b_l3.reshape(1, 1),
       T2a_b.reshape(1, 1), T2b_b.reshape(1, 1), lin2_W,
       lin2_b.reshape(1, 1), lin1_W, lin1_b.reshape(1, 1)],
      [_rows(128)] + [_rows(1)] * 10
      + [_full((128, 1)), _full((1, 1)), _full((1, 1)), _full((1, 1)),
         _full((2, 1)), _full((1, 1)), _full((2, 1)), _full((1, 1))],
      jax.ShapeDtypeStruct((VP, 1), F32),
      _rows(1),
  )
  return out[:N]


# speedup vs baseline: 19.8359x; 1.0396x over previous
import functools

import jax
import jax.numpy as jnp
from jax import lax
from jax.experimental import pallas as pl
from jax.experimental.pallas import tpu as pltpu
from jax.experimental.pallas import tpu_sc as plsc

N = 50000
E = 800000
NC = 2
NS = 16
NW = NC * NS
VP = 50048
EW = E // NW
STR = VP // NS

F32 = jnp.float32


def _mesh():
  return plsc.VectorSubcoreMesh(core_axis_name="c", subcore_axis_name="s")


@functools.lru_cache(maxsize=None)
def _make_hop(D, chunk):
  iters = EW // chunk
  vec = D == 0
  tab_s = (VP,) if vec else (VP, D)
  row_s = (chunk,) if vec else (chunk, D)
  if vec:
    out_t = jax.ShapeDtypeStruct((NC * VP,), F32)
  else:
    out_t = (jax.ShapeDtypeStruct((VP, D), F32),
             jax.ShapeDtypeStruct((VP, D), F32))

  @functools.partial(
      pl.kernel,
      mesh=_mesh(),
      compiler_params=pltpu.CompilerParams(use_tc_tiling_on_sc=False),
      out_type=out_t,
      scratch_types=[
          pltpu.VMEM((chunk,), jnp.int32),
          pltpu.VMEM((chunk,), jnp.int32),
          pltpu.VMEM((chunk,), jnp.int32),
          pltpu.VMEM((chunk,), jnp.int32),
          pltpu.VMEM(row_s, F32),
          pltpu.VMEM(row_s, F32),
          pltpu.VMEM_SHARED(tab_s, F32),
          pltpu.SemaphoreType.DMA,
          pltpu.SemaphoreType.DMA,
          pltpu.SemaphoreType.DMA,
      ],
  )
  def hop(table, src, dst, zeros, *rest):
    if vec:
      (out, src_v0, src_v1, dst_v0, dst_v1, rows_v0, rows_v1, acc,
       sem_i0, sem_i1, sem_g) = rest
    else:
      (out0, out1, src_v0, src_v1, dst_v0, dst_v1, rows_v0, rows_v1, acc,
       sem_i0, sem_i1, sem_g) = rest
    srcb = (src_v0, src_v1)
    dstb = (dst_v0, dst_v1)
    rowb = (rows_v0, rows_v1)
    semi = (sem_i0, sem_i1)
    c = lax.axis_index("c")
    s = lax.axis_index("s")
    r0 = s * STR
    sub = STR if chunk >= STR else 184
    for j in range(STR // sub):
      o = r0 + j * sub
      pltpu.sync_copy(zeros.at[pl.ds(o, sub)], rows_v0.at[pl.ds(0, sub)])
      pltpu.sync_copy(rows_v0.at[pl.ds(0, sub)], acc.at[pl.ds(o, sub)])
    plsc.subcore_barrier()
    base = (s * NC + c) * EW

    def start_idx(t):
      b = t % 2
      off = base + t * chunk
      d1 = pltpu.async_copy(src.at[pl.ds(off, chunk)], srcb[b], semi[b])
      d2 = pltpu.async_copy(dst.at[pl.ds(off, chunk)], dstb[b], semi[b])
      return (d1, d2)

    idx_d = [None] * iters
    gat_d = [None] * iters
    idx_d[0] = start_idx(0)
    idx_d[0][0].wait()
    idx_d[0][1].wait()
    gat_d[0] = pltpu.async_copy(table.at[srcb[0]], rowb[0], sem_g)
    for t in range(iters):
      cur = t % 2
      nxt = 1 - cur
      if t + 1 < iters:
        idx_d[t + 1] = start_idx(t + 1)
      gat_d[t].wait()
      if t + 1 < iters:
        idx_d[t + 1][0].wait()
        idx_d[t + 1][1].wait()
        gat_d[t + 1] = pltpu.async_copy(table.at[srcb[nxt]], rowb[nxt],
                                        sem_g)
      pltpu.sync_copy(rowb[cur], acc.at[dstb[cur]], add=True)
    plsc.subcore_barrier()
    for j in range(STR // sub):
      o = r0 + j * sub
      pltpu.sync_copy(acc.at[pl.ds(o, sub)], rows_v0.at[pl.ds(0, sub)])
      if vec:
        pltpu.sync_copy(rows_v0.at[pl.ds(0, sub)],
                        out.at[pl.ds(c * VP + o, sub)])
      else:
        @pl.when(c == 0)
        def _():
          pltpu.sync_copy(rows_v0.at[pl.ds(0, sub)], out0.at[pl.ds(o, sub)])

        @pl.when(c == 1)
        def _():
          pltpu.sync_copy(rows_v0.at[pl.ds(0, sub)], out1.at[pl.ds(o, sub)])

  return hop


VP2 = VP + 16
VSTR = 3136
CH1 = 2000
EPT = E // NS


def _make_chains():
  @functools.partial(
      pl.kernel,
      mesh=_mesh(),
      compiler_params=pltpu.CompilerParams(use_tc_tiling_on_sc=False),
      out_type=(jax.ShapeDtypeStruct((VP,), F32),
                jax.ShapeDtypeStruct((VP,), F32),
                jax.ShapeDtypeStruct((VP,), F32)),
      scratch_types=[
          pltpu.VMEM((CH1,), jnp.int32),
          pltpu.VMEM((CH1,), jnp.int32),
          pltpu.VMEM((CH1,), F32),
          pltpu.VMEM((VSTR,), F32),
          pltpu.VMEM((VSTR,), F32),
          pltpu.VMEM((VSTR,), F32),
          pltpu.VMEM((VSTR,), F32),
          pltpu.VMEM_SHARED((VP2,), F32),
          pltpu.VMEM_SHARED((VP2,), F32),
          pltpu.SemaphoreType.DMA,
      ],
  )
  def chains(sab, src, dst, dis2v, zrs, qb_o, q3_o, qa_o,
             src_v, dst_v, rows_v, t_v, sab_v, dis2_v, zeros_v,
             table, acc, sem):
    c = lax.axis_index("c")
    s = lax.axis_index("s")
    o = s * STR
    pltpu.sync_copy(zrs, zeros_v)
    pltpu.sync_copy(dis2v.at[pl.ds(o, VSTR)], dis2_v)

    def init_table(row):
      pltpu.sync_copy(sab.at[row, pl.ds(o, VSTR)], t_v)
      pltpu.sync_copy(t_v, table.at[pl.ds(o, VSTR)])

    def zero_acc():
      pltpu.sync_copy(zeros_v.at[pl.ds(0, STR)], acc.at[pl.ds(o, STR)])

    def edge_pass():
      ebase = s * EPT

      def body(t, _):
        off = ebase + t * CH1
        pltpu.sync_copy(src.at[pl.ds(off, CH1)], src_v)
        pltpu.sync_copy(dst.at[pl.ds(off, CH1)], dst_v)
        pltpu.async_copy(table.at[src_v], rows_v, sem).wait()
        pltpu.sync_copy(rows_v, acc.at[dst_v], add=True)
        return _

      lax.fori_loop(0, EPT // CH1, body, None)

    def advance(row):
      pltpu.sync_copy(acc.at[pl.ds(o, VSTR)], t_v)
      pltpu.sync_copy(sab.at[row, pl.ds(o, VSTR)], sab_v)

      def vbody(i, _):
        sl = pl.ds(i * 16, 16)
        t_v[sl] = sab_v[sl] + dis2_v[sl] * t_v[sl]
        return _

      lax.fori_loop(0, VSTR // 16, vbody, None)
      pltpu.sync_copy(t_v, table.at[pl.ds(o, VSTR)])

    def copy_out(dst_hbm):
      pltpu.sync_copy(acc.at[pl.ds(o, STR)], t_v.at[pl.ds(0, STR)])
      pltpu.sync_copy(t_v.at[pl.ds(0, STR)], dst_hbm.at[pl.ds(o, STR)])

    @pl.when(c == 0)
    def _():
      init_table(0)
      zero_acc()
      plsc.subcore_barrier()
      for h in range(9):
        edge_pass()
        plsc.subcore_barrier()
        if h < 8:
          advance(h + 1)
          plsc.subcore_barrier()
          zero_acc()
        else:
          copy_out(qb_o)
          init_table(9)
          zero_acc()
        plsc.subcore_barrier()
      edge_pass()
      plsc.subcore_barrier()
      copy_out(q3_o)

    @pl.when(c == 1)
    def _():
      init_table(10)
      zero_acc()
      plsc.subcore_barrier()
      for h in range(3):
        edge_pass()
        plsc.subcore_barrier()
        if h < 2:
          advance(11 + h)
          plsc.subcore_barrier()
          zero_acc()
        else:
          copy_out(qa_o)
        plsc.subcore_barrier()

  return chains


B = 544
G = VP // B
FR = VP // 128


def _rows(k):
  return pl.BlockSpec((B, k), lambda i: (i, 0))


def _full(shape):
  return pl.BlockSpec(shape, lambda i: (0,) * len(shape))


def _tc(body, ins, in_specs, out_shapes, out_specs, grid=None):
  return pl.pallas_call(
      body,
      grid=(G,) if grid is None else grid,
      in_specs=in_specs,
      out_specs=out_specs,
      out_shape=out_shapes,
  )(*ins)


def _prep_body(d0, d1, x, dis_o, dis2_o, invmd_o, sx_o):
  deg = d0[...] + d1[...]
  md = jnp.maximum(deg, 1.0)
  dis = jnp.where(deg > 0, lax.rsqrt(md), 0.0)
  dis_o[...] = dis
  dis2_o[...] = dis * dis
  invmd_o[...] = 1.0 / md
  sx_o[...] = x[...] * dis


def _sage1_body(q0, q1, x, invmd, wl, wr, b, x1_o):
  mean = (q0[...] + q1[...]) * invmd[...]
  x1_o[...] = jnp.maximum(
      jnp.dot(mean, wl[...], preferred_element_type=F32)
      + jnp.dot(x[...], wr[...], preferred_element_type=F32) + b[...], 0.0)


def _sage2_body(*refs):
  qs = refs[:16]
  x1, invmd, wl, wr, b, wl3, x12_o, y3_o = refs[16:]
  w = wl[...]
  im = invmd[...]
  acc = jnp.dot(x1[...], wr[...], preferred_element_type=F32) + b[...]
  for p in range(8):
    u = (qs[2 * p][...] + qs[2 * p + 1][...]) * im
    acc = acc + jnp.dot(u, w[16 * p:16 * p + 16, :],
                        preferred_element_type=F32)
  x12 = jnp.maximum(acc, 0.0)
  x12_o[...] = x12
  y3_o[...] = jnp.dot(x12, wl3[...], preferred_element_type=F32)


def _tagscale_body(q0, q1, dis2, tbl_o):
  tbl_o[...] = dis2[...] * (q0[...] + q1[...])


def _tagdense_body(*refs):
  (qt, x, dis, wa, ba, wb, bb, w2a, w2b,
   za_o, sa_o, zb_o, sb_o) = (refs[:18], *refs[18:])
  ua = jnp.zeros((B, 8), F32)
  ub = jnp.zeros((B, 8), F32)
  for k in range(9):
    u = qt[2 * k][...] + qt[2 * k + 1][...]
    if k < 3:
      ua = ua + jnp.dot(u, wa[k + 1], preferred_element_type=F32)
    ub = ub + jnp.dot(u, wb[k + 1], preferred_element_type=F32)
  d = dis[...]
  x2 = jnp.maximum(
      jnp.dot(x[...], wa[0], preferred_element_type=F32) + d * ua + ba[...],
      0.0)
  x3 = jnp.maximum(
      jnp.dot(x[...], wb[0], preferred_element_type=F32) + d * ub + bb[...],
      0.0)
  za_o[...] = jnp.dot(x2, w2a[...][:, :1], preferred_element_type=F32)
  sa_o[...] = jnp.dot(x2, w2a[...][:, 1:], preferred_element_type=F32) * d
  zb_o[...] = jnp.dot(x3, w2b[...][:, :1], preferred_element_type=F32)
  sb_o[...] = jnp.dot(x3, w2b[...][:, 1:], preferred_element_type=F32) * d


def _hglue_body(sa, q0, q1, dis2, tbl_o):
  tbl_o[...] = sa[...] + dis2[...] * (q0[...] + q1[...])


def _flat(a):
  return a.reshape(FR, 128)


def _final_body(x12, q3, qa, qb, za, zb, dis, invmd,
                wr3, bl3, b2a, b2b, l2w, l2b, l1w, l1b, out_o):
  d = dis[...]
  x13 = jnp.maximum(
      q3[...] * invmd[...] + bl3[...]
      + jnp.dot(x12[...], wr3[...], preferred_element_type=F32), 0.0)
  x2f = jnp.maximum(za[...] + d * qa[...] + b2a[...], 0.0)
  x3f = jnp.maximum(zb[...] + d * qb[...] + b2b[...], 0.0)
  w2 = l2w[...]
  x23 = jnp.maximum(x2f * w2[0:1, :] + x3f * w2[1:2, :] + l2b[...], 0.0)
  w1 = l1w[...]
  out_o[...] = jnp.maximum(x13 * w1[0:1, :] + x23 * w1[1:2, :] + l1b[...],
                           0.0)


def kernel(x, edge_index, W_l1, b_l1, W_r1, W_l2, b_l2, W_r2, W_l3, b_l3,
           W_r3, T1a_W, T1a_b, T2a_W, T2a_b, T1b_W, T1b_b, T2b_W, T2b_b,
           lin2_W, lin2_b, lin1_W, lin1_b):
  xp = jnp.pad(x, ((0, VP - N), (0, 0)))
  xp16 = jnp.pad(x, ((0, VP - N), (0, 8)))
  src = edge_index[0]
  dst = edge_index[1]
  z1 = jnp.zeros((VP,), F32)
  z16 = jnp.zeros((VP, 16), F32)

  ones_vp = jnp.ones((VP,), F32)

  hop1 = _make_hop(0, 5000)
  hop16 = _make_hop(16, 1000)

  degp = hop1(ones_vp, src, dst, z1).reshape(NC, VP)
  d0 = degp[0].reshape(VP, 1)
  d1 = degp[1].reshape(VP, 1)

  dis, dis2, invmd, sx = _tc(
      _prep_body,
      [d0, d1, xp16],
      [_rows(1), _rows(1), _rows(16)],
      [jax.ShapeDtypeStruct((VP, 1), F32)] * 3
      + [jax.ShapeDtypeStruct((VP, 16), F32)],
      [_rows(1), _rows(1), _rows(1), _rows(16)],
  )

  wl1p = jnp.pad(W_l1, ((0, 8), (0, 0)))
  wr1p = jnp.pad(W_r1, ((0, 8), (0, 0)))
  q8 = hop16(xp16, src, dst, z16)
  x1 = _tc(
      _sage1_body,
      [q8[0], q8[1], xp16, invmd, wl1p, wr1p, b_l1.reshape(1, 128)],
      [_rows(16), _rows(16), _rows(16), _rows(1), _full((16, 128)),
       _full((16, 128)), _full((1, 128))],
      jax.ShapeDtypeStruct((VP, 128), F32),
      _rows(128),
  )

  x1t = x1.reshape(VP, 8, 16).transpose(1, 0, 2)
  qs = [hop16(x1t[p], src, dst, z16) for p in range(8)]
  qflat = [q for pair in qs for q in pair]
  x12, y3 = _tc(
      _sage2_body,
      qflat + [x1, invmd, W_l2, W_r2, b_l2.reshape(1, 128), W_l3],
      [_rows(16)] * 16 + [_rows(128), _rows(1), _full((128, 128)),
                          _full((128, 128)), _full((1, 128)),
                          _full((128, 1))],
      [jax.ShapeDtypeStruct((VP, 128), F32),
       jax.ShapeDtypeStruct((VP, 1), F32)],
      [_rows(128), _rows(1)],
  )


  qt = []
  tbl = sx
  for k in range(9):
    qk = hop16(tbl, src, dst, z16)
    qt.append(qk)
    if k < 8:
      tbl = _tc(
          _tagscale_body,
          [qk[0], qk[1], dis2],
          [_rows(16), _rows(16), _rows(1)],
          jax.ShapeDtypeStruct((VP, 16), F32),
          _rows(16),
      )

  w2a = T2a_W[..., 0].T
  w2b = T2b_W[..., 0].T
  qt_flat = []
  for qk in qt:
    qt_flat += [qk[0], qk[1]]
  za, sa, zb, sb = _tc(
      _tagdense_body,
      qt_flat + [xp16, dis, jnp.pad(T1a_W, ((0, 0), (0, 8), (0, 0))),
                 T1a_b.reshape(1, 8),
                 jnp.pad(T1b_W, ((0, 0), (0, 8), (0, 0))),
                 T1b_b.reshape(1, 8), w2a, w2b],
      [_rows(16)] * 18 + [_rows(16), _rows(1), _full((4, 16, 8)),
                         _full((1, 8)), _full((10, 16, 8)), _full((1, 8)),
                         _full((8, 4)), _full((8, 10))],
      [jax.ShapeDtypeStruct((VP, 1), F32),
       jax.ShapeDtypeStruct((VP, 3), F32),
       jax.ShapeDtypeStruct((VP, 1), F32),
       jax.ShapeDtypeStruct((VP, 9), F32)],
      [_rows(1), _rows(3), _rows(1), _rows(9)],
  )

  pad2 = VP2 - VP
  sabrows = (
      [sb[:, 8 - j] for j in range(9)]
      + [y3[:, 0]]
      + [sa[:, 2], sa[:, 1], sa[:, 0]]
  )
  sab = jnp.pad(jnp.stack(sabrows), ((0, 0), (0, pad2)))
  dis2v2 = jnp.pad(dis2[:, 0], ((0, pad2),))
  z3136 = jnp.zeros((VSTR,), F32)
  chains = _make_chains()
  qb, q3, qa = chains(sab, src, dst, dis2v2, z3136)

  out = _tc(
      _final_body,
      [x12, q3.reshape(VP, 1), qa.reshape(VP, 1), qb.reshape(VP, 1),
       za, zb, dis, invmd, W_r3, b_l3.reshape(1, 1),
       T2a_b.reshape(1, 1), T2b_b.reshape(1, 1), lin2_W,
       lin2_b.reshape(1, 1), lin1_W, lin1_b.reshape(1, 1)],
      [_rows(128)] + [_rows(1)] * 7
      + [_full((128, 1)), _full((1, 1)), _full((1, 1)), _full((1, 1)),
         _full((2, 1)), _full((1, 1)), _full((2, 1)), _full((1, 1))],
      jax.ShapeDtypeStruct((VP, 1), F32),
      _rows(1),
  )
  return out[:N]


# speedup vs baseline: 22.5709x; 1.1379x over previous
import functools

import jax
import jax.numpy as jnp
from jax import lax
from jax.experimental import pallas as pl
from jax.experimental.pallas import tpu as pltpu
from jax.experimental.pallas import tpu_sc as plsc

N = 50000
E = 800000
NC = 2
NS = 16
NW = NC * NS
VP = 50048
EW = E // NW
STR = VP // NS

F32 = jnp.float32


def _mesh():
  return plsc.VectorSubcoreMesh(core_axis_name="c", subcore_axis_name="s")


@functools.lru_cache(maxsize=None)
def _make_hop(D, chunk):
  iters = EW // chunk
  vec = D == 0
  tab_s = (VP,) if vec else (VP, D)
  row_s = (chunk,) if vec else (chunk, D)
  if vec:
    out_t = jax.ShapeDtypeStruct((NC * VP,), F32)
  else:
    out_t = (jax.ShapeDtypeStruct((VP, D), F32),
             jax.ShapeDtypeStruct((VP, D), F32))

  @functools.partial(
      pl.kernel,
      mesh=_mesh(),
      compiler_params=pltpu.CompilerParams(use_tc_tiling_on_sc=False),
      out_type=out_t,
      scratch_types=[
          pltpu.VMEM((chunk,), jnp.int32),
          pltpu.VMEM((chunk,), jnp.int32),
          pltpu.VMEM((chunk,), jnp.int32),
          pltpu.VMEM((chunk,), jnp.int32),
          pltpu.VMEM(row_s, F32),
          pltpu.VMEM(row_s, F32),
          pltpu.VMEM_SHARED(tab_s, F32),
          pltpu.SemaphoreType.DMA,
          pltpu.SemaphoreType.DMA,
          pltpu.SemaphoreType.DMA,
      ],
  )
  def hop(table, src, dst, zeros, *rest):
    if vec:
      (out, src_v0, src_v1, dst_v0, dst_v1, rows_v0, rows_v1, acc,
       sem_i0, sem_i1, sem_g) = rest
    else:
      (out0, out1, src_v0, src_v1, dst_v0, dst_v1, rows_v0, rows_v1, acc,
       sem_i0, sem_i1, sem_g) = rest
    srcb = (src_v0, src_v1)
    dstb = (dst_v0, dst_v1)
    rowb = (rows_v0, rows_v1)
    semi = (sem_i0, sem_i1)
    c = lax.axis_index("c")
    s = lax.axis_index("s")
    r0 = s * STR
    sub = STR if chunk >= STR else 184
    for j in range(STR // sub):
      o = r0 + j * sub
      pltpu.sync_copy(zeros.at[pl.ds(o, sub)], rows_v0.at[pl.ds(0, sub)])
      pltpu.sync_copy(rows_v0.at[pl.ds(0, sub)], acc.at[pl.ds(o, sub)])
    plsc.subcore_barrier()
    base = (s * NC + c) * EW

    def start_idx(t):
      b = t % 2
      off = base + t * chunk
      d1 = pltpu.async_copy(src.at[pl.ds(off, chunk)], srcb[b], semi[b])
      d2 = pltpu.async_copy(dst.at[pl.ds(off, chunk)], dstb[b], semi[b])
      return (d1, d2)

    idx_d = [None] * iters
    gat_d = [None] * iters
    idx_d[0] = start_idx(0)
    idx_d[0][0].wait()
    idx_d[0][1].wait()
    gat_d[0] = pltpu.async_copy(table.at[srcb[0]], rowb[0], sem_g)
    for t in range(iters):
      cur = t % 2
      nxt = 1 - cur
      if t + 1 < iters:
        idx_d[t + 1] = start_idx(t + 1)
      gat_d[t].wait()
      if t + 1 < iters:
        idx_d[t + 1][0].wait()
        idx_d[t + 1][1].wait()
        gat_d[t + 1] = pltpu.async_copy(table.at[srcb[nxt]], rowb[nxt],
                                        sem_g)
      pltpu.sync_copy(rowb[cur], acc.at[dstb[cur]], add=True)
    plsc.subcore_barrier()
    for j in range(STR // sub):
      o = r0 + j * sub
      pltpu.sync_copy(acc.at[pl.ds(o, sub)], rows_v0.at[pl.ds(0, sub)])
      if vec:
        pltpu.sync_copy(rows_v0.at[pl.ds(0, sub)],
                        out.at[pl.ds(c * VP + o, sub)])
      else:
        @pl.when(c == 0)
        def _():
          pltpu.sync_copy(rows_v0.at[pl.ds(0, sub)], out0.at[pl.ds(o, sub)])

        @pl.when(c == 1)
        def _():
          pltpu.sync_copy(rows_v0.at[pl.ds(0, sub)], out1.at[pl.ds(o, sub)])

  return hop


VP2 = VP + 16
VSTR = 3136
CH1 = 2000
EPT = E // NS


def _make_chains():
  @functools.partial(
      pl.kernel,
      mesh=_mesh(),
      compiler_params=pltpu.CompilerParams(use_tc_tiling_on_sc=False),
      out_type=(jax.ShapeDtypeStruct((VP,), F32),
                jax.ShapeDtypeStruct((VP,), F32),
                jax.ShapeDtypeStruct((VP,), F32)),
      scratch_types=[
          pltpu.VMEM((CH1,), jnp.int32),
          pltpu.VMEM((CH1,), jnp.int32),
          pltpu.VMEM((CH1,), F32),
          pltpu.VMEM((VSTR,), F32),
          pltpu.VMEM((VSTR,), F32),
          pltpu.VMEM((VSTR,), F32),
          pltpu.VMEM((VSTR,), F32),
          pltpu.VMEM_SHARED((VP2,), F32),
          pltpu.VMEM_SHARED((VP2,), F32),
          pltpu.SemaphoreType.DMA,
      ],
  )
  def chains(sab, src, dst, dis2v, zrs, qb_o, q3_o, qa_o,
             src_v, dst_v, rows_v, t_v, sab_v, dis2_v, zeros_v,
             table, acc, sem):
    c = lax.axis_index("c")
    s = lax.axis_index("s")
    o = s * STR
    pltpu.sync_copy(zrs, zeros_v)
    pltpu.sync_copy(dis2v.at[pl.ds(o, VSTR)], dis2_v)

    def init_table(row):
      pltpu.sync_copy(sab.at[row, pl.ds(o, VSTR)], t_v)
      pltpu.sync_copy(t_v, table.at[pl.ds(o, VSTR)])

    def zero_acc():
      pltpu.sync_copy(zeros_v.at[pl.ds(0, STR)], acc.at[pl.ds(o, STR)])

    def edge_pass():
      ebase = s * EPT

      def body(t, _):
        off = ebase + t * CH1
        pltpu.sync_copy(src.at[pl.ds(off, CH1)], src_v)
        pltpu.sync_copy(dst.at[pl.ds(off, CH1)], dst_v)
        pltpu.async_copy(table.at[src_v], rows_v, sem).wait()
        pltpu.sync_copy(rows_v, acc.at[dst_v], add=True)
        return _

      lax.fori_loop(0, EPT // CH1, body, None)

    def advance(row):
      pltpu.sync_copy(acc.at[pl.ds(o, VSTR)], t_v)
      pltpu.sync_copy(sab.at[row, pl.ds(o, VSTR)], sab_v)

      def vbody(i, _):
        sl = pl.ds(i * 16, 16)
        t_v[sl] = sab_v[sl] + dis2_v[sl] * t_v[sl]
        return _

      lax.fori_loop(0, VSTR // 16, vbody, None)
      pltpu.sync_copy(t_v, table.at[pl.ds(o, VSTR)])

    def copy_out(dst_hbm):
      pltpu.sync_copy(acc.at[pl.ds(o, STR)], t_v.at[pl.ds(0, STR)])
      pltpu.sync_copy(t_v.at[pl.ds(0, STR)], dst_hbm.at[pl.ds(o, STR)])

    @pl.when(c == 0)
    def _():
      init_table(0)
      zero_acc()
      plsc.subcore_barrier()
      for h in range(9):
        edge_pass()
        plsc.subcore_barrier()
        if h < 8:
          advance(h + 1)
          plsc.subcore_barrier()
          zero_acc()
        else:
          copy_out(qb_o)
          init_table(9)
          zero_acc()
        plsc.subcore_barrier()
      edge_pass()
      plsc.subcore_barrier()
      copy_out(q3_o)

    @pl.when(c == 1)
    def _():
      init_table(10)
      zero_acc()
      plsc.subcore_barrier()
      for h in range(3):
        edge_pass()
        plsc.subcore_barrier()
        if h < 2:
          advance(11 + h)
          plsc.subcore_barrier()
          zero_acc()
        else:
          copy_out(qa_o)
        plsc.subcore_barrier()

  return chains


B = 544
G = VP // B
FR = VP // 128


def _rows(k):
  return pl.BlockSpec((B, k), lambda i: (i, 0))


def _full(shape):
  return pl.BlockSpec(shape, lambda i: (0,) * len(shape))


def _tc(body, ins, in_specs, out_shapes, out_specs, grid=None):
  return pl.pallas_call(
      body,
      grid=(G,) if grid is None else grid,
      in_specs=in_specs,
      out_specs=out_specs,
      out_shape=out_shapes,
  )(*ins)


def _prep_body(d0, d1, x, dis_o, dis2_o, invmd_o, sx_o, d2r_o):
  deg = d0[...] + d1[...]
  md = jnp.maximum(deg, 1.0)
  dis = jnp.where(deg > 0, lax.rsqrt(md), 0.0)
  dis_o[...] = dis
  dis2 = dis * dis
  dis2_o[...] = dis2
  invmd_o[...] = 1.0 / md
  sx_o[...] = x[...] * dis
  d2r_o[...] = jnp.broadcast_to(dis2, (B, 16))


def _sage1_body(q0, q1, x, invmd, wl, wr, b, x1_o):
  mean = (q0[...] + q1[...]) * invmd[...]
  x1_o[...] = jnp.maximum(
      jnp.dot(mean, wl[...], preferred_element_type=F32)
      + jnp.dot(x[...], wr[...], preferred_element_type=F32) + b[...], 0.0)


def _sage2_body(*refs):
  qs = refs[:16]
  x1, invmd, wl, wr, b, wl3, x12_o, y3_o = refs[16:]
  w = wl[...]
  im = invmd[...]
  acc = jnp.dot(x1[...], wr[...], preferred_element_type=F32) + b[...]
  for p in range(8):
    u = (qs[2 * p][...] + qs[2 * p + 1][...]) * im
    acc = acc + jnp.dot(u, w[16 * p:16 * p + 16, :],
                        preferred_element_type=F32)
  x12 = jnp.maximum(acc, 0.0)
  x12_o[...] = x12
  y3_o[...] = jnp.dot(x12, wl3[...], preferred_element_type=F32)


def _tagscale_body(q0, q1, d2r, tbl_o):
  tbl_o[...] = d2r[...] * (q0[...] + q1[...])


def _tagdense_body(*refs):
  (qt, x, dis, wa, ba, wb, bb, w2a, w2b,
   za_o, sa_o, zb_o, sb_o) = (refs[:18], *refs[18:])
  ua = jnp.zeros((B, 8), F32)
  ub = jnp.zeros((B, 8), F32)
  for k in range(9):
    u = qt[2 * k][...] + qt[2 * k + 1][...]
    if k < 3:
      ua = ua + jnp.dot(u, wa[k + 1], preferred_element_type=F32)
    ub = ub + jnp.dot(u, wb[k + 1], preferred_element_type=F32)
  d = dis[...]
  x2 = jnp.maximum(
      jnp.dot(x[...], wa[0], preferred_element_type=F32) + d * ua + ba[...],
      0.0)
  x3 = jnp.maximum(
      jnp.dot(x[...], wb[0], preferred_element_type=F32) + d * ub + bb[...],
      0.0)
  za_o[...] = jnp.dot(x2, w2a[...][:, :1], preferred_element_type=F32)
  sa_o[...] = jnp.dot(x2, w2a[...][:, 1:], preferred_element_type=F32) * d
  zb_o[...] = jnp.dot(x3, w2b[...][:, :1], preferred_element_type=F32)
  sb_o[...] = jnp.dot(x3, w2b[...][:, 1:], preferred_element_type=F32) * d


def _hglue_body(sa, q0, q1, dis2, tbl_o):
  tbl_o[...] = sa[...] + dis2[...] * (q0[...] + q1[...])


def _flat(a):
  return a.reshape(FR, 128)


def _final_body(x12, q3, qa, qb, za, zb, dis, invmd,
                wr3, bl3, b2a, b2b, l2w, l2b, l1w, l1b, out_o):
  d = dis[...]
  x13 = jnp.maximum(
      q3[...] * invmd[...] + bl3[...]
      + jnp.dot(x12[...], wr3[...], preferred_element_type=F32), 0.0)
  x2f = jnp.maximum(za[...] + d * qa[...] + b2a[...], 0.0)
  x3f = jnp.maximum(zb[...] + d * qb[...] + b2b[...], 0.0)
  w2 = l2w[...]
  x23 = jnp.maximum(x2f * w2[0:1, :] + x3f * w2[1:2, :] + l2b[...], 0.0)
  w1 = l1w[...]
  out_o[...] = jnp.maximum(x13 * w1[0:1, :] + x23 * w1[1:2, :] + l1b[...],
                           0.0)


def kernel(x, edge_index, W_l1, b_l1, W_r1, W_l2, b_l2, W_r2, W_l3, b_l3,
           W_r3, T1a_W, T1a_b, T2a_W, T2a_b, T1b_W, T1b_b, T2b_W, T2b_b,
           lin2_W, lin2_b, lin1_W, lin1_b):
  xp = jnp.pad(x, ((0, VP - N), (0, 0)))
  xp16 = jnp.pad(x, ((0, VP - N), (0, 8)))
  src = edge_index[0]
  dst = edge_index[1]
  z1 = jnp.zeros((VP,), F32)
  z16 = jnp.zeros((VP, 16), F32)

  ones_vp = jnp.ones((VP,), F32)

  hop1 = _make_hop(0, 5000)
  hop16 = _make_hop(16, 1000)

  degp = hop1(ones_vp, src, dst, z1).reshape(NC, VP)
  d0 = degp[0].reshape(VP, 1)
  d1 = degp[1].reshape(VP, 1)

  dis, dis2, invmd, sx, d2r16 = _tc(
      _prep_body,
      [d0, d1, xp16],
      [_rows(1), _rows(1), _rows(16)],
      [jax.ShapeDtypeStruct((VP, 1), F32)] * 3
      + [jax.ShapeDtypeStruct((VP, 16), F32)] * 2,
      [_rows(1), _rows(1), _rows(1), _rows(16), _rows(16)],
  )
  d2r_flat = d2r16.reshape(VP * 16 // 128, 128)

  wl1p = jnp.pad(W_l1, ((0, 8), (0, 0)))
  wr1p = jnp.pad(W_r1, ((0, 8), (0, 0)))
  q8 = hop16(xp16, src, dst, z16)
  x1 = _tc(
      _sage1_body,
      [q8[0], q8[1], xp16, invmd, wl1p, wr1p, b_l1.reshape(1, 128)],
      [_rows(16), _rows(16), _rows(16), _rows(1), _full((16, 128)),
       _full((16, 128)), _full((1, 128))],
      jax.ShapeDtypeStruct((VP, 128), F32),
      _rows(128),
  )

  x1t = x1.reshape(VP, 8, 16).transpose(1, 0, 2)
  qs = [hop16(x1t[p], src, dst, z16) for p in range(8)]
  qflat = [q for pair in qs for q in pair]
  x12, y3 = _tc(
      _sage2_body,
      qflat + [x1, invmd, W_l2, W_r2, b_l2.reshape(1, 128), W_l3],
      [_rows(16)] * 16 + [_rows(128), _rows(1), _full((128, 128)),
                          _full((128, 128)), _full((1, 128)),
                          _full((128, 1))],
      [jax.ShapeDtypeStruct((VP, 128), F32),
       jax.ShapeDtypeStruct((VP, 1), F32)],
      [_rows(128), _rows(1)],
  )


  qt = []
  tbl = sx
  for k in range(9):
    qk = hop16(tbl, src, dst, z16)
    qt.append(qk)
    if k < 8:
      fs = VP * 16 // 128
      tbl = _tc(
          _tagscale_body,
          [qk[0].reshape(fs, 128), qk[1].reshape(fs, 128), d2r_flat],
          [_full((fs, 128))] * 3,
          jax.ShapeDtypeStruct((fs, 128), F32),
          _full((fs, 128)),
          grid=(1,),
      ).reshape(VP, 16)

  w2a = T2a_W[..., 0].T
  w2b = T2b_W[..., 0].T
  qt_flat = []
  for qk in qt:
    qt_flat += [qk[0], qk[1]]
  za, sa, zb, sb = _tc(
      _tagdense_body,
      qt_flat + [xp16, dis, jnp.pad(T1a_W, ((0, 0), (0, 8), (0, 0))),
                 T1a_b.reshape(1, 8),
                 jnp.pad(T1b_W, ((0, 0), (0, 8), (0, 0))),
                 T1b_b.reshape(1, 8), w2a, w2b],
      [_rows(16)] * 18 + [_rows(16), _rows(1), _full((4, 16, 8)),
                         _full((1, 8)), _full((10, 16, 8)), _full((1, 8)),
                         _full((8, 4)), _full((8, 10))],
      [jax.ShapeDtypeStruct((VP, 1), F32),
       jax.ShapeDtypeStruct((VP, 3), F32),
       jax.ShapeDtypeStruct((VP, 1), F32),
       jax.ShapeDtypeStruct((VP, 9), F32)],
      [_rows(1), _rows(3), _rows(1), _rows(9)],
  )

  pad2 = VP2 - VP
  sabrows = (
      [sb[:, 8 - j] for j in range(9)]
      + [y3[:, 0]]
      + [sa[:, 2], sa[:, 1], sa[:, 0]]
  )
  sab = jnp.pad(jnp.stack(sabrows), ((0, 0), (0, pad2)))
  dis2v2 = jnp.pad(dis2[:, 0], ((0, pad2),))
  z3136 = jnp.zeros((VSTR,), F32)
  chains = _make_chains()
  qb, q3, qa = chains(sab, src, dst, dis2v2, z3136)

  out = _tc(
      _final_body,
      [x12, q3.reshape(VP, 1), qa.reshape(VP, 1), qb.reshape(VP, 1),
       za, zb, dis, invmd, W_r3, b_l3.reshape(1, 1),
       T2a_b.reshape(1, 1), T2b_b.reshape(1, 1), lin2_W,
       lin2_b.reshape(1, 1), lin1_W, lin1_b.reshape(1, 1)],
      [_rows(128)] + [_rows(1)] * 7
      + [_full((128, 1)), _full((1, 1)), _full((1, 1)), _full((1, 1)),
         _full((2, 1)), _full((1, 1)), _full((2, 1)), _full((1, 1))],
      jax.ShapeDtypeStruct((VP, 1), F32),
      _rows(1),
  )
  return out[:N]
